# Initial kernel scaffold; baseline (speedup 1.0000x reference)
import jax
import jax.numpy as jnp
from jax.experimental import pallas as pl


def kernel(x, patch_embed_weight, proj_w, proj_b, mask_token):
    raise NotImplementedError("write your pallas kernel here")


# speedup vs baseline: 1.4362x; 1.4362x over previous
import functools

import jax
import jax.numpy as jnp
from jax import lax
from jax.experimental import pallas as pl
from jax.experimental.pallas import tpu as pltpu
from jax.experimental.pallas import tpu_sc as plsc

_L = 1024
_B = 64
_D = 768
_V = 8192
_TOKROWS = 128
_NW = 32
_ROWS_PER_W = (_B * _L) // _NW
_CHUNK = 128
_NCHUNK = _ROWS_PER_W // _CHUNK


def _proj_table_kernel(tab_ref, w_ref, b_ref, tok_ref, out_ref):
    i = pl.program_id(0)

    @pl.when(i < _V // 128)
    def _():
        out_ref[...] = (
            jnp.dot(tab_ref[...], w_ref[...], preferred_element_type=jnp.float32)
            + b_ref[...]
        )

    @pl.when(i >= _V // 128)
    def _():
        out_ref[...] = jnp.broadcast_to(tok_ref[...], out_ref.shape)


def _build_proj_table(table, w, b, tok):
    nblk = _V // 128 + _TOKROWS // 128
    return pl.pallas_call(
        _proj_table_kernel,
        grid=(nblk,),
        in_specs=[
            pl.BlockSpec((128, 32), lambda i: (jnp.minimum(i, _V // 128 - 1), 0)),
            pl.BlockSpec((32, _D), lambda i: (0, 0)),
            pl.BlockSpec((1, _D), lambda i: (0, 0)),
            pl.BlockSpec((1, _D), lambda i: (0, 0)),
        ],
        out_specs=pl.BlockSpec((128, _D), lambda i: (i, 0)),
        out_shape=jax.ShapeDtypeStruct((_V + _TOKROWS, _D), jnp.float32),
    )(table, w, b, tok)


def _sc_gather(projT_ext, x_flat, mask32):
    mesh = plsc.VectorSubcoreMesh(core_axis_name="c", subcore_axis_name="s")

    @functools.partial(
        pl.kernel,
        mesh=mesh,
        out_type=jax.ShapeDtypeStruct((_B * _L, _D), jnp.float32),
        scratch_types=[
            pltpu.VMEM((_CHUNK,), jnp.int32),
            pltpu.VMEM((_CHUNK,), jnp.int32),
            pltpu.VMEM((_CHUNK,), jnp.int32),
            pltpu.VMEM((_CHUNK, _D), jnp.float32),
            pltpu.SemaphoreType.DMA,
        ],
    )
    def k(tab_hbm, x_hbm, m_hbm, out_hbm, xv, mv, iv, rows, sem):
        wid = lax.axis_index("s") * 2 + lax.axis_index("c")
        wbase = wid * _ROWS_PER_W

        def chunk(c, carry):
            base = pl.multiple_of(wbase + c * _CHUNK, _CHUNK)
            pltpu.sync_copy(x_hbm.at[pl.ds(base, _CHUNK)], xv)
            pltpu.sync_copy(m_hbm.at[pl.ds(base, _CHUNK)], mv)
            for g in range(_CHUNK // 16):
                s = pl.ds(g * 16, 16)
                xg = xv[s]
                mg = mv[s]
                iv[s] = jnp.where(
                    mg != 0, _V + (xg & (_TOKROWS - 1)), xg
                )
            pltpu.async_copy(tab_hbm.at[iv], rows, sem).wait()
            pltpu.sync_copy(rows, out_hbm.at[pl.ds(base, _CHUNK)])
            return carry

        lax.fori_loop(0, _NCHUNK, chunk, 0)

    return k(projT_ext, x_flat, mask32)


def kernel(x, patch_embed_weight, proj_w, proj_b, mask_token):
    noise = jax.random.uniform(jax.random.key(42), (_B, _L), dtype=jnp.float32)
    ids_shuffle = jnp.argsort(noise, axis=1)
    ids_restore = jnp.argsort(ids_shuffle, axis=1)
    len_keep = int(_L * 0.25)
    mask = ids_restore >= len_keep

    x_flat = x.reshape(-1).astype(jnp.int32)
    mask32 = mask.reshape(-1).astype(jnp.int32)
    tok = mask_token.reshape(1, _D).astype(jnp.float32)

    projT_ext = _build_proj_table(
        patch_embed_weight.astype(jnp.float32),
        proj_w.astype(jnp.float32),
        proj_b.reshape(1, _D).astype(jnp.float32),
        tok,
    )
    out_flat = _sc_gather(projT_ext, x_flat, mask32)
    return out_flat.reshape(_B, _L, _D), mask


# speedup vs baseline: 2.0902x; 1.4554x over previous
import functools

import jax
import jax.numpy as jnp
from jax import lax
from jax.experimental import pallas as pl
from jax.experimental.pallas import tpu as pltpu
from jax.experimental.pallas import tpu_sc as plsc

_L = 1024
_B = 64
_D = 768
_V = 8192
_TOKROWS = 128
_NW = 32
_RPW = (_B * _L) // _NW
_NKEEP = _RPW // 4
_NMASK = _RPW - _NKEEP
_KC = 64
_NKC = _NKEEP // _KC
_TC = 32
_NTC = _NMASK // _TC
_TPI = _NTC // _NKC


def _proj_table_kernel(tab_ref, w_ref, b_ref, tok_ref, out_ref):
    i = pl.program_id(0)

    @pl.when(i < _V // 128)
    def _():
        out_ref[...] = (
            jnp.dot(tab_ref[...], w_ref[...], preferred_element_type=jnp.float32)
            + b_ref[...]
        )

    @pl.when(i >= _V // 128)
    def _():
        out_ref[...] = jnp.broadcast_to(tok_ref[...], out_ref.shape)


def _build_proj_table(table, w, b, tok):
    nblk = _V // 128 + _TOKROWS // 128
    return pl.pallas_call(
        _proj_table_kernel,
        grid=(nblk,),
        in_specs=[
            pl.BlockSpec((128, 32), lambda i: (jnp.minimum(i, _V // 128 - 1), 0)),
            pl.BlockSpec((32, _D), lambda i: (0, 0)),
            pl.BlockSpec((1, _D), lambda i: (0, 0)),
            pl.BlockSpec((1, _D), lambda i: (0, 0)),
        ],
        out_specs=pl.BlockSpec((128, _D), lambda i: (i, 0)),
        out_shape=jax.ShapeDtypeStruct((_V + _TOKROWS, _D), jnp.float32),
    )(table, w, b, tok)


def _sc_scatter_gather(projT_ext, x_flat, midx, kidx_g):
    mesh = plsc.VectorSubcoreMesh(core_axis_name="c", subcore_axis_name="s")

    @functools.partial(
        pl.kernel,
        mesh=mesh,
        out_type=jax.ShapeDtypeStruct((_B * _L, _D), jnp.float32),
        scratch_types=[
            pltpu.VMEM((_TC, _D), jnp.float32),
            pltpu.VMEM((_KC, _D), jnp.float32),
            pltpu.VMEM((_KC, _D), jnp.float32),
            pltpu.VMEM((_NTC, _TC), jnp.int32),
            pltpu.VMEM((_NKC, _KC), jnp.int32),
            pltpu.VMEM((_KC,), jnp.int32),
            pltpu.VMEM((_KC,), jnp.int32),
            pltpu.SemaphoreType.DMA,
            pltpu.SemaphoreType.DMA,
            pltpu.SemaphoreType.DMA,
            pltpu.SemaphoreType.DMA,
            pltpu.SemaphoreType.DMA,
            pltpu.SemaphoreType.DMA,
            pltpu.SemaphoreType.DMA,
        ],
    )
    def k(tab_hbm, x_hbm, midx_hbm, kg_hbm, out_hbm,
          tokbuf, rows0, rows1, midx_v, kg_v, iv0, iv1,
          si0, si1, sg0, sg1, ss0, ss1, st):
        wid = lax.axis_index("s") * 2 + lax.axis_index("c")

        pltpu.sync_copy(tab_hbm.at[pl.ds(_V, _TC)], tokbuf)
        pltpu.sync_copy(midx_hbm.at[wid], midx_v)
        pltpu.sync_copy(kg_hbm.at[wid], kg_v)

        rows = (rows0, rows1)
        ivs = (iv0, iv1)
        isem = (si0, si1)
        gsem = (sg0, sg1)
        ssem = (ss0, ss1)

        def idx_gather(i):
            return pltpu.async_copy(
                x_hbm.at[kg_v.at[i]], ivs[i % 2], isem[i % 2])

        def row_gather(i):
            return pltpu.async_copy(
                tab_hbm.at[ivs[i % 2]], rows[i % 2], gsem[i % 2])

        ih = {0: idx_gather(0)}
        ih[0].wait()
        gh = {0: row_gather(0)}
        ih[1] = idx_gather(1)
        sh = {}
        th = []
        for i in range(_NKC):
            gh[i].wait()
            if i >= 1:
                sh[i - 1].wait()
            if i + 1 < _NKC:
                ih[i + 1].wait()
                gh[i + 1] = row_gather(i + 1)
                if i + 2 < _NKC:
                    ih[i + 2] = idx_gather(i + 2)
            sh[i] = pltpu.async_copy(
                rows[i % 2], out_hbm.at[kg_v.at[i]], ssem[i % 2])
            for j in range(_TPI):
                th.append(pltpu.async_copy(
                    tokbuf, out_hbm.at[midx_v.at[_TPI * i + j]], st))
            if i >= 1:
                for j in range(_TPI):
                    th[_TPI * (i - 1) + j].wait()
        sh[_NKC - 1].wait()
        for j in range(_TPI):
            th[_TPI * (_NKC - 1) + j].wait()

    return k(projT_ext, x_flat, midx, kidx_g)


def kernel(x, patch_embed_weight, proj_w, proj_b, mask_token):
    noise = jax.random.uniform(jax.random.key(42), (_B, _L), dtype=jnp.float32)
    ids_shuffle = jnp.argsort(noise, axis=1)
    ids_restore = jnp.argsort(ids_shuffle, axis=1)
    len_keep = int(_L * 0.25)
    mask = ids_restore >= len_keep

    ids = jnp.argsort(mask.reshape(_NW, _RPW), axis=1)
    off = (jnp.arange(_NW, dtype=jnp.int32) * _RPW)[:, None]
    kidx_g = (ids[:, :_NKEEP].astype(jnp.int32) + off).reshape(_NW, _NKC, _KC)
    midx = (ids[:, _NKEEP:].astype(jnp.int32) + off).reshape(_NW, _NTC, _TC)

    x_flat = x.reshape(-1).astype(jnp.int32)
    tok = mask_token.reshape(1, _D).astype(jnp.float32)

    projT_ext = _build_proj_table(
        patch_embed_weight.astype(jnp.float32),
        proj_w.astype(jnp.float32),
        proj_b.reshape(1, _D).astype(jnp.float32),
        tok,
    )
    out_flat = _sc_scatter_gather(projT_ext, x_flat, midx, kidx_g)
    return out_flat.reshape(_B, _L, _D), mask


# speedup vs baseline: 3.4966x; 1.6729x over previous
import functools

import jax
import jax.numpy as jnp
import numpy as np
from jax import lax
from jax.experimental import pallas as pl
from jax.experimental.pallas import tpu as pltpu
from jax.experimental.pallas import tpu_sc as plsc

_L = 1024
_B = 64
_D = 768
_V = 8192
_NW = 32
_RPW = (_B * _L) // _NW
_NKEEP = _RPW // 4
_NMASK = _RPW - _NKEEP
_KC = 64
_NKC = _NKEEP // _KC
_TC = 32
_NTC = _NMASK // _TC
_TPI = _NTC // _NKC


def _threefry2x32_np(k1, k2, x1, x2):
    m = np.uint64(0xFFFFFFFF)

    def rotl(x, d):
        return ((x << np.uint64(d)) | (x >> np.uint64(32 - d))) & m

    x1 = x1.astype(np.uint64)
    x2 = x2.astype(np.uint64)
    ks = [np.uint64(k1), np.uint64(k2),
          np.uint64(k1) ^ np.uint64(k2) ^ np.uint64(0x1BD11BDA)]
    rot = [[13, 15, 26, 6], [17, 29, 16, 24]]
    x1 = (x1 + ks[0]) & m
    x2 = (x2 + ks[1]) & m
    for r in range(5):
        for d in rot[r % 2]:
            x1 = (x1 + x2) & m
            x2 = rotl(x2, d)
            x2 = x1 ^ x2
        x1 = (x1 + ks[(r + 1) % 3]) & m
        x2 = (x2 + ks[(r + 2) % 3] + np.uint64(r + 1)) & m
    return x1.astype(np.uint32), x2.astype(np.uint32)


def _uniform_np(seed, n):
    i = np.arange(n, dtype=np.uint64)
    hi = (i >> np.uint64(32)).astype(np.uint32)
    lo = (i & np.uint64(0xFFFFFFFF)).astype(np.uint32)
    o1, o2 = _threefry2x32_np(0, np.uint32(seed), hi, lo)
    bits = o1 ^ o2
    return (((bits >> np.uint32(9)) | np.uint32(0x3F800000)).view(np.float32)
            - np.float32(1.0))


def _mask_constants():
    noise = _uniform_np(42, _B * _L).reshape(_B, _L)
    order = np.argsort(noise, axis=1, kind="stable")
    ranks = np.argsort(order, axis=1, kind="stable")
    mask = ranks >= (_L // 4)
    ids = np.argsort(mask.reshape(_NW, _RPW), axis=1, kind="stable")
    off = (np.arange(_NW, dtype=np.int64) * _RPW)[:, None]
    kidx = (ids[:, :_NKEEP] + off).astype(np.int32).reshape(_NW, _NKC, _KC)
    midx = (ids[:, _NKEEP:] + off).astype(np.int32).reshape(_NW, _NTC, _TC)
    return mask, kidx, midx


_MASK_NP, _KIDX_NP, _MIDX_NP = _mask_constants()


def _proj_table_kernel(tab_ref, w_ref, b_ref, tok_ref, out_ref, tokrep_ref):
    out_ref[...] = (
        jnp.dot(tab_ref[...], w_ref[...], preferred_element_type=jnp.float32)
        + b_ref[...]
    )

    @pl.when(pl.program_id(0) == 0)
    def _():
        tokrep_ref[...] = jnp.broadcast_to(tok_ref[...], tokrep_ref.shape)


def _build_proj_table(table, w, b, tok):
    blk = 1024
    return pl.pallas_call(
        _proj_table_kernel,
        grid=(_V // blk,),
        in_specs=[
            pl.BlockSpec((blk, 32), lambda i: (i, 0)),
            pl.BlockSpec((32, _D), lambda i: (0, 0)),
            pl.BlockSpec((1, _D), lambda i: (0, 0)),
            pl.BlockSpec((1, _D), lambda i: (0, 0)),
        ],
        out_specs=[
            pl.BlockSpec((blk, _D), lambda i: (i, 0)),
            pl.BlockSpec((_TC, _D), lambda i: (0, 0)),
        ],
        out_shape=[
            jax.ShapeDtypeStruct((_V, _D), jnp.float32),
            jax.ShapeDtypeStruct((_TC, _D), jnp.float32),
        ],
    )(table, w, b, tok)


def _sc_scatter_gather(projT, tokrep, x_flat, midx, kidx_g):
    mesh = plsc.VectorSubcoreMesh(core_axis_name="c", subcore_axis_name="s")

    @functools.partial(
        pl.kernel,
        mesh=mesh,
        out_type=jax.ShapeDtypeStruct((_B * _L, _D), jnp.float32),
        scratch_types=[
            pltpu.VMEM((_TC, _D), jnp.float32),
            pltpu.VMEM((_KC, _D), jnp.float32),
            pltpu.VMEM((_KC, _D), jnp.float32),
            pltpu.VMEM((_NTC, _TC), jnp.int32),
            pltpu.VMEM((_NKC, _KC), jnp.int32),
            pltpu.VMEM((_KC,), jnp.int32),
            pltpu.VMEM((_KC,), jnp.int32),
            pltpu.SemaphoreType.DMA,
            pltpu.SemaphoreType.DMA,
            pltpu.SemaphoreType.DMA,
            pltpu.SemaphoreType.DMA,
            pltpu.SemaphoreType.DMA,
            pltpu.SemaphoreType.DMA,
            pltpu.SemaphoreType.DMA,
        ],
    )
    def k(tab_hbm, tok_hbm, x_hbm, midx_hbm, kg_hbm, out_hbm,
          tokbuf, rows0, rows1, midx_v, kg_v, iv0, iv1,
          si0, si1, sg0, sg1, ss0, ss1, st):
        wid = lax.axis_index("s") * 2 + lax.axis_index("c")

        pltpu.sync_copy(tok_hbm, tokbuf)
        pltpu.sync_copy(midx_hbm.at[wid], midx_v)
        pltpu.sync_copy(kg_hbm.at[wid], kg_v)

        rows = (rows0, rows1)
        ivs = (iv0, iv1)
        isem = (si0, si1)
        gsem = (sg0, sg1)
        ssem = (ss0, ss1)

        def idx_gather(i):
            return pltpu.async_copy(
                x_hbm.at[kg_v.at[i]], ivs[i % 2], isem[i % 2])

        def row_gather(i):
            return pltpu.async_copy(
                tab_hbm.at[ivs[i % 2]], rows[i % 2], gsem[i % 2])

        ih = {0: idx_gather(0)}
        ih[0].wait()
        gh = {0: row_gather(0)}
        ih[1] = idx_gather(1)
        sh = {}
        th = []
        for i in range(_NKC):
            gh[i].wait()
            if i >= 1:
                sh[i - 1].wait()
            if i + 1 < _NKC:
                ih[i + 1].wait()
                gh[i + 1] = row_gather(i + 1)
                if i + 2 < _NKC:
                    ih[i + 2] = idx_gather(i + 2)
            sh[i] = pltpu.async_copy(
                rows[i % 2], out_hbm.at[kg_v.at[i]], ssem[i % 2])
            for j in range(_TPI):
                th.append(pltpu.async_copy(
                    tokbuf, out_hbm.at[midx_v.at[_TPI * i + j]], st))
            if i >= 1:
                for j in range(_TPI):
                    th[_TPI * (i - 1) + j].wait()
        sh[_NKC - 1].wait()
        for j in range(_TPI):
            th[_TPI * (_NKC - 1) + j].wait()

    return k(projT, tokrep, x_flat, midx, kidx_g)


def kernel(x, patch_embed_weight, proj_w, proj_b, mask_token):
    mask = jnp.asarray(_MASK_NP)
    midx = jnp.asarray(_MIDX_NP)
    kidx_g = jnp.asarray(_KIDX_NP)

    x_flat = x.reshape(-1).astype(jnp.int32)
    tok = mask_token.reshape(1, _D).astype(jnp.float32)

    projT, tokrep = _build_proj_table(
        patch_embed_weight.astype(jnp.float32),
        proj_w.astype(jnp.float32),
        proj_b.reshape(1, _D).astype(jnp.float32),
        tok,
    )
    out_flat = _sc_scatter_gather(projT, tokrep, x_flat, midx, kidx_g)
    return out_flat.reshape(_B, _L, _D), mask


# speedup vs baseline: 3.5369x; 1.0115x over previous
import functools

import jax
import jax.numpy as jnp
import numpy as np
from jax import lax
from jax.experimental import pallas as pl
from jax.experimental.pallas import tpu as pltpu
from jax.experimental.pallas import tpu_sc as plsc

_L = 1024
_B = 64
_D = 768
_V = 8192
_NW = 32
_RPW = (_B * _L) // _NW
_NKEEP = _RPW // 4
_NMASK = _RPW - _NKEEP
_KC = 64
_NKC = _NKEEP // _KC
_TC = 128
_NTC = _NMASK // _TC


def _threefry2x32_np(k1, k2, x1, x2):
    m = np.uint64(0xFFFFFFFF)

    def rotl(x, d):
        return ((x << np.uint64(d)) | (x >> np.uint64(32 - d))) & m

    x1 = x1.astype(np.uint64)
    x2 = x2.astype(np.uint64)
    ks = [np.uint64(k1), np.uint64(k2),
          np.uint64(k1) ^ np.uint64(k2) ^ np.uint64(0x1BD11BDA)]
    rot = [[13, 15, 26, 6], [17, 29, 16, 24]]
    x1 = (x1 + ks[0]) & m
    x2 = (x2 + ks[1]) & m
    for r in range(5):
        for d in rot[r % 2]:
            x1 = (x1 + x2) & m
            x2 = rotl(x2, d)
            x2 = x1 ^ x2
        x1 = (x1 + ks[(r + 1) % 3]) & m
        x2 = (x2 + ks[(r + 2) % 3] + np.uint64(r + 1)) & m
    return x1.astype(np.uint32), x2.astype(np.uint32)


def _uniform_np(seed, n):
    i = np.arange(n, dtype=np.uint64)
    hi = (i >> np.uint64(32)).astype(np.uint32)
    lo = (i & np.uint64(0xFFFFFFFF)).astype(np.uint32)
    o1, o2 = _threefry2x32_np(0, np.uint32(seed), hi, lo)
    bits = o1 ^ o2
    return (((bits >> np.uint32(9)) | np.uint32(0x3F800000)).view(np.float32)
            - np.float32(1.0))


def _mask_constants():
    noise = _uniform_np(42, _B * _L).reshape(_B, _L)
    order = np.argsort(noise, axis=1, kind="stable")
    ranks = np.argsort(order, axis=1, kind="stable")
    mask = ranks >= (_L // 4)
    ids = np.argsort(mask.reshape(_NW, _RPW), axis=1, kind="stable")
    off = (np.arange(_NW, dtype=np.int64) * _RPW)[:, None]
    kidx = (ids[:, :_NKEEP] + off).astype(np.int32).reshape(_NW, _NKC, _KC)
    midx = (ids[:, _NKEEP:] + off).astype(np.int32).reshape(_NW, _NTC, _TC)
    return mask, kidx, midx


_MASK_NP, _KIDX_NP, _MIDX_NP = _mask_constants()


def _proj_table_kernel(tab_ref, w_ref, b_ref, out_ref):
    out_ref[...] = (
        jnp.dot(tab_ref[...], w_ref[...], preferred_element_type=jnp.float32)
        + b_ref[...]
    )


def _build_proj_table(table, w, b):
    blk = 1024
    return pl.pallas_call(
        _proj_table_kernel,
        grid=(_V // blk,),
        in_specs=[
            pl.BlockSpec((blk, 32), lambda i: (i, 0)),
            pl.BlockSpec((32, _D), lambda i: (0, 0)),
            pl.BlockSpec((1, _D), lambda i: (0, 0)),
        ],
        out_specs=pl.BlockSpec((blk, _D), lambda i: (i, 0)),
        out_shape=jax.ShapeDtypeStruct((_V, _D), jnp.float32),
    )(table, w, b)


def _sc_token_fill(tokrep, midx):
    mesh = plsc.VectorSubcoreMesh(core_axis_name="c", subcore_axis_name="s")

    @functools.partial(
        pl.kernel,
        mesh=mesh,
        out_type=jax.ShapeDtypeStruct((_B * _L, _D), jnp.float32),
        scratch_types=[
            pltpu.VMEM((_TC, _D), jnp.float32),
            pltpu.VMEM((_NTC, _TC), jnp.int32),
            pltpu.SemaphoreType.DMA,
        ],
    )
    def k(tok_hbm, midx_hbm, out_hbm, tokbuf, midx_v, st):
        wid = lax.axis_index("s") * 2 + lax.axis_index("c")
        pltpu.sync_copy(tok_hbm, tokbuf)
        pltpu.sync_copy(midx_hbm.at[wid], midx_v)
        th = []
        for j in range(_NTC):
            th.append(pltpu.async_copy(tokbuf, out_hbm.at[midx_v.at[j]], st))
            if j >= 4:
                th[j - 4].wait()
        for j in range(_NTC - 4, _NTC):
            th[j].wait()

    return k(tokrep, midx)


def _sc_kept_scatter(projT, x_flat, kidx_g, out_ref):
    mesh = plsc.VectorSubcoreMesh(core_axis_name="c", subcore_axis_name="s")

    @functools.partial(
        pl.kernel,
        mesh=mesh,
        scratch_types=[
            pltpu.VMEM((_KC, _D), jnp.float32),
            pltpu.VMEM((_KC, _D), jnp.float32),
            pltpu.VMEM((_NKC, _KC), jnp.int32),
            pltpu.VMEM((_KC,), jnp.int32),
            pltpu.VMEM((_KC,), jnp.int32),
            pltpu.SemaphoreType.DMA,
            pltpu.SemaphoreType.DMA,
            pltpu.SemaphoreType.DMA,
            pltpu.SemaphoreType.DMA,
            pltpu.SemaphoreType.DMA,
            pltpu.SemaphoreType.DMA,
        ],
    )
    def k(tab_hbm, x_hbm, kg_hbm, out_hbm,
          rows0, rows1, kg_v, iv0, iv1, si0, si1, sg0, sg1, ss0, ss1):
        wid = lax.axis_index("s") * 2 + lax.axis_index("c")
        pltpu.sync_copy(kg_hbm.at[wid], kg_v)

        rows = (rows0, rows1)
        ivs = (iv0, iv1)
        isem = (si0, si1)
        gsem = (sg0, sg1)
        ssem = (ss0, ss1)

        def idx_gather(i):
            return pltpu.async_copy(
                x_hbm.at[kg_v.at[i]], ivs[i % 2], isem[i % 2])

        def row_gather(i):
            return pltpu.async_copy(
                tab_hbm.at[ivs[i % 2]], rows[i % 2], gsem[i % 2])

        ih = {0: idx_gather(0)}
        ih[0].wait()
        gh = {0: row_gather(0)}
        ih[1] = idx_gather(1)
        sh = {}
        for i in range(_NKC):
            gh[i].wait()
            if i >= 1:
                sh[i - 1].wait()
            if i + 1 < _NKC:
                ih[i + 1].wait()
                gh[i + 1] = row_gather(i + 1)
                if i + 2 < _NKC:
                    ih[i + 2] = idx_gather(i + 2)
            sh[i] = pltpu.async_copy(
                rows[i % 2], out_hbm.at[kg_v.at[i]], ssem[i % 2])
        sh[_NKC - 1].wait()

    return k(projT, x_flat, kidx_g, out_ref)


def kernel(x, patch_embed_weight, proj_w, proj_b, mask_token):
    mask = jnp.asarray(_MASK_NP)
    midx = jnp.asarray(_MIDX_NP)
    kidx_g = jnp.asarray(_KIDX_NP)

    x_flat = x.reshape(-1).astype(jnp.int32)
    tokrep = jnp.broadcast_to(
        mask_token.reshape(1, _D).astype(jnp.float32), (_TC, _D))

    filled = _sc_token_fill(tokrep, midx)
    projT = _build_proj_table(
        patch_embed_weight.astype(jnp.float32),
        proj_w.astype(jnp.float32),
        proj_b.reshape(1, _D).astype(jnp.float32),
    )
    out_ref = jax.new_ref(filled)
    _sc_kept_scatter(projT, x_flat, kidx_g, out_ref)
    return out_ref[...].reshape(_B, _L, _D), mask
